# baseline (speedup 1.0000x reference)
import functools

import jax
import jax.numpy as jnp
from jax import lax
from jax.experimental import pallas as pl
from jax.experimental.pallas import tpu as pltpu
from jax.experimental.pallas import tpu_sc as plsc

BB = 256
L = 16


def kernel(lookup, table):
    B, T = lookup.shape
    V, D = table.shape
    DP = 128

    info = plsc.get_sparse_core_info()
    NW = info.num_cores * info.num_subcores
    NBLK = B // BB
    units_total = T * NBLK
    upw = units_total // NW

    idx_flat = jnp.transpose(lookup).reshape(-1).astype(jnp.int32)
    table_p = jnp.pad(table, ((0, 0), (0, DP - D)))

    mesh = plsc.VectorSubcoreMesh(core_axis_name="c", subcore_axis_name="s")

    @functools.partial(
        pl.kernel,
        mesh=mesh,
        out_type=jax.ShapeDtypeStruct((T, D, B), jnp.float32),
        scratch_types=[
            *[pltpu.VMEM((BB,), jnp.int32) for _ in range(2)],
            *[pltpu.VMEM((BB, DP), jnp.float32) for _ in range(2)],
            *[pltpu.VMEM((D, BB), jnp.float32) for _ in range(2)],
            *[pltpu.SemaphoreType.DMA for _ in range(4)],
        ],
        compiler_params=pltpu.CompilerParams(
            use_tc_tiling_on_sc=True, needs_layout_passes=False
        ),
    )
    def gather_kernel(table_hbm, idx_hbm, out_hbm, *bufs):
        idxv = bufs[0:2]
        rows = bufs[2:4]
        obuf = bufs[4:6]
        gsem = bufs[6:8]
        osem = bufs[8:10]

        wid = lax.axis_index("s") * info.num_cores + lax.axis_index("c")
        blk = wid // 2
        t_base = (wid % 2) * upw
        b0 = blk * BB

        rowvecs = [lax.iota(jnp.int32, L) + bc * L for bc in range(BB // L)]

        def fire(u, p):
            t = t_base + u
            pltpu.sync_copy(idx_hbm.at[pl.ds(t * B + b0, BB)], idxv[p])
            for c in range(BB // 128):
                pltpu.async_copy(
                    table_hbm.at[idxv[p].at[pl.ds(c * 128, 128)]],
                    rows[p].at[pl.ds(c * 128, 128)],
                    gsem[p],
                )

        def drain_gather(p):
            for c in range(BB // 128):
                pltpu.make_async_copy(
                    table_hbm.at[pl.ds(0, 128)],
                    rows[p].at[pl.ds(c * 128, 128)],
                    gsem[p],
                ).wait()

        def transpose(p):
            def dbody(d, _):
                for bc in range(BB // L):
                    v = plsc.load_gather(
                        rows[p],
                        [rowvecs[bc], jnp.full((L,), d, jnp.int32)],
                    )
                    obuf[p][d, pl.ds(bc * L, L)] = v
                return _
            lax.fori_loop(0, D, dbody, None)

        def start_wb(u, p):
            t = t_base + u
            pltpu.async_copy(
                obuf[p], out_hbm.at[t, :, pl.ds(b0, BB)], osem[p]
            )

        def wait_wb(p):
            pltpu.make_async_copy(
                out_hbm.at[0, :, pl.ds(0, BB)], obuf[p], osem[p]
            ).wait()

        fire(0, 0)

        def unit(u, p, do_fire, do_wait_wb):
            drain_gather(p)
            if do_fire:
                fire(u + 1, 1 - p)
            if do_wait_wb:
                wait_wb(p)
            transpose(p)
            start_wb(u, p)

        unit(0, 0, True, False)
        unit(1, 1, True, False)

        def body(k, _):
            u = 2 + 2 * k
            unit(u, 0, True, True)
            unit(u + 1, 1, True, True)
            return _

        lax.fori_loop(0, (upw - 4) // 2, body, None)

        unit(upw - 2, 0, True, True)
        unit(upw - 1, 1, False, True)
        wait_wb(0)
        wait_wb(1)

    out = gather_kernel(table_p, idx_flat)
    return jnp.transpose(out, (2, 0, 1))


# speedup vs baseline: 1.2052x; 1.2052x over previous
import functools

import jax
import jax.numpy as jnp
from jax import lax
from jax.experimental import pallas as pl
from jax.experimental.pallas import tpu as pltpu
from jax.experimental.pallas import tpu_sc as plsc

BB = 256
L = 16


def kernel(lookup, table):
    B, T = lookup.shape
    V, D = table.shape
    DP = 128

    info = plsc.get_sparse_core_info()
    NW = info.num_cores * info.num_subcores
    NBLK = B // BB
    units_total = T * NBLK
    upw = units_total // NW

    idx_flat = jnp.transpose(lookup).reshape(-1).astype(jnp.int32)
    table_p = jnp.pad(table, ((0, 0), (0, DP - D)))

    mesh = plsc.VectorSubcoreMesh(core_axis_name="c", subcore_axis_name="s")

    @functools.partial(
        pl.kernel,
        mesh=mesh,
        out_type=jax.ShapeDtypeStruct((T, D, B), jnp.float32),
        scratch_types=[
            *[pltpu.VMEM((BB,), jnp.int32) for _ in range(2)],
            *[pltpu.VMEM((BB, DP), jnp.float32) for _ in range(2)],
            *[pltpu.VMEM((D, BB), jnp.float32) for _ in range(2)],
            *[pltpu.SemaphoreType.DMA for _ in range(4)],
        ],
        compiler_params=pltpu.CompilerParams(
            use_tc_tiling_on_sc=True, needs_layout_passes=False
        ),
    )
    def gather_kernel(table_hbm, idx_hbm, out_hbm, *bufs):
        idxv = bufs[0:2]
        rows = bufs[2:4]
        obuf = bufs[4:6]
        gsem = bufs[6:8]
        osem = bufs[8:10]

        wid = lax.axis_index("s") * info.num_cores + lax.axis_index("c")
        blk = wid // 2
        t_base = (wid % 2) * upw
        b0 = blk * BB

        iot = lax.iota(jnp.int32, L)

        def fire(u, p):
            t = t_base + u
            pltpu.sync_copy(idx_hbm.at[pl.ds(t * B + b0, BB)], idxv[p])
            for c in range(BB // 128):
                pltpu.async_copy(
                    table_hbm.at[idxv[p].at[pl.ds(c * 128, 128)]],
                    rows[p].at[pl.ds(c * 128, 128)],
                    gsem[p],
                )

        def drain_gather(p):
            for c in range(BB // 128):
                pltpu.make_async_copy(
                    table_hbm.at[pl.ds(0, 128)],
                    rows[p].at[pl.ds(c * 128, 128)],
                    gsem[p],
                ).wait()

        def transpose(p):
            def bbody(bb, _):
                base = bb * L
                for c in range(D // L):
                    colidx = iot + c * L
                    for i in range(L):
                        v = rows[p][base + i, pl.ds(c * L, L)]
                        plsc.store_scatter(
                            obuf[p],
                            [colidx, jnp.full((L,), base + i, jnp.int32)],
                            v,
                        )
                return _
            lax.fori_loop(0, BB // L, bbody, None)

        def start_wb(u, p):
            t = t_base + u
            pltpu.async_copy(
                obuf[p], out_hbm.at[t, :, pl.ds(b0, BB)], osem[p]
            )

        def wait_wb(p):
            pltpu.make_async_copy(
                out_hbm.at[0, :, pl.ds(0, BB)], obuf[p], osem[p]
            ).wait()

        fire(0, 0)

        def unit(u, p, do_fire, do_wait_wb):
            drain_gather(p)
            if do_fire:
                fire(u + 1, 1 - p)
            if do_wait_wb:
                wait_wb(p)
            transpose(p)
            start_wb(u, p)

        unit(0, 0, True, False)
        unit(1, 1, True, False)

        def body(k, _):
            u = 2 + 2 * k
            unit(u, 0, True, True)
            unit(u + 1, 1, True, True)
            return _

        lax.fori_loop(0, (upw - 4) // 2, body, None)

        unit(upw - 2, 0, True, True)
        unit(upw - 1, 1, False, True)
        wait_wb(0)
        wait_wb(1)

    out = gather_kernel(table_p, idx_flat)
    return jnp.transpose(out, (2, 0, 1))


# speedup vs baseline: 1.2333x; 1.0234x over previous
import functools

import jax
import jax.numpy as jnp
from jax import lax
from jax.experimental import pallas as pl
from jax.experimental.pallas import tpu as pltpu
from jax.experimental.pallas import tpu_sc as plsc

BB = 256
L = 16
NI = 4


def kernel(lookup, table):
    B, T = lookup.shape
    V, D = table.shape
    DP = 128

    info = plsc.get_sparse_core_info()
    NW = info.num_cores * info.num_subcores
    NBLK = B // BB
    upw = (T * NBLK) // NW

    idx_flat = jnp.transpose(lookup).reshape(-1).astype(jnp.int32)
    table_p = jnp.pad(table, ((0, 0), (0, DP - D)))

    mesh = plsc.VectorSubcoreMesh(core_axis_name="c", subcore_axis_name="s")

    @functools.partial(
        pl.kernel,
        mesh=mesh,
        out_type=jax.ShapeDtypeStruct((T, D, B), jnp.float32),
        scratch_types=[
            *[pltpu.VMEM((BB,), jnp.int32) for _ in range(NI)],
            *[pltpu.VMEM((BB, DP), jnp.float32) for _ in range(2)],
            *[pltpu.VMEM((D, BB), jnp.float32) for _ in range(2)],
            *[pltpu.SemaphoreType.DMA for _ in range(NI)],
            *[pltpu.SemaphoreType.DMA for _ in range(4)],
        ],
        compiler_params=pltpu.CompilerParams(
            use_tc_tiling_on_sc=True, needs_layout_passes=False
        ),
    )
    def gather_kernel(table_hbm, idx_hbm, out_hbm, *bufs):
        idxq = bufs[0:NI]
        rows = bufs[NI:NI + 2]
        obuf = bufs[NI + 2:NI + 4]
        isem = bufs[NI + 4:2 * NI + 4]
        gsem = bufs[2 * NI + 4:2 * NI + 6]
        osem = bufs[2 * NI + 6:2 * NI + 8]

        wid = lax.axis_index("s") * info.num_cores + lax.axis_index("c")
        blk = wid // 2
        t_base = (wid % 2) * upw
        b0 = blk * BB

        iot = lax.iota(jnp.int32, L)
        colv = [iot + c * L for c in range(D // L)]

        def start_idx(u, q):
            t = t_base + u
            pltpu.async_copy(
                idx_hbm.at[pl.ds(t * B + b0, BB)], idxq[q], isem[q]
            )

        def wait_idx(q):
            pltpu.make_async_copy(
                idx_hbm.at[pl.ds(0, BB)], idxq[q], isem[q]
            ).wait()

        def fire_g(q, p):
            for c in range(BB // 128):
                pltpu.async_copy(
                    table_hbm.at[idxq[q].at[pl.ds(c * 128, 128)]],
                    rows[p].at[pl.ds(c * 128, 128)],
                    gsem[p],
                )

        def drain_gather(p):
            for c in range(BB // 128):
                pltpu.make_async_copy(
                    table_hbm.at[pl.ds(0, 128)],
                    rows[p].at[pl.ds(c * 128, 128)],
                    gsem[p],
                ).wait()

        def transpose(p):
            def bbody(bb, _):
                base = bb * L
                for i in range(L):
                    vs = [rows[p][base + i, pl.ds(c * L, L)]
                          for c in range(D // L)]
                    ridx = jnp.full((L,), base + i, jnp.int32)
                    for c in range(D // L):
                        plsc.store_scatter(obuf[p], [colv[c], ridx], vs[c])
                return _
            lax.fori_loop(0, BB // L, bbody, None)

        def start_wb(u, p):
            t = t_base + u
            pltpu.async_copy(
                obuf[p], out_hbm.at[t, :, pl.ds(b0, BB)], osem[p]
            )

        def wait_wb(p):
            pltpu.make_async_copy(
                out_hbm.at[0, :, pl.ds(0, BB)], obuf[p], osem[p]
            ).wait()

        def unit_ops(u, p, do_start_idx, do_fire, do_wait_wb, q2, q1):
            if do_start_idx:
                start_idx(u + 2, q2)
            if do_fire:
                wait_idx(q1)
                fire_g(q1, 1 - p)
            drain_gather(p)
            if do_wait_wb:
                wait_wb(p)
            transpose(p)
            start_wb(u, p)

        start_idx(0, 0)
        start_idx(1, 1)
        wait_idx(0)
        fire_g(0, 0)
        unit_ops(0, 0, True, True, False, 2, 1)
        unit_ops(1, 1, True, True, False, 3, 2)

        def body(m, _):
            u0 = 2 + 4 * m
            for s in range(4):
                unit_ops(u0 + s, s % 2, True, True, True, s, (3 + s) % 4)
            return _

        lax.fori_loop(0, (upw - 4) // 4, body, None)

        unit_ops(upw - 2, 0, False, True, True, 0, (upw - 1) % NI)
        unit_ops(upw - 1, 1, False, False, True, 0, 0)
        wait_wb(0)
        wait_wb(1)

    out = gather_kernel(table_p, idx_flat)
    return jnp.transpose(out, (2, 0, 1))


# speedup vs baseline: 2.0779x; 1.6848x over previous
import functools

import jax
import jax.numpy as jnp
from jax import lax
from jax.experimental import pallas as pl
from jax.experimental.pallas import tpu as pltpu
from jax.experimental.pallas import tpu_sc as plsc

BB = 256
L = 16
NI = 4


def kernel(lookup, table):
    B, T = lookup.shape
    V, D = table.shape
    DP = 128

    info = plsc.get_sparse_core_info()
    NW = info.num_cores * info.num_subcores
    NBLK = B // BB
    upw = (T * NBLK) // NW

    idx_flat = jnp.transpose(lookup).reshape(-1).astype(jnp.int32)
    table_p = jnp.pad(table, ((0, 0), (0, DP - D)))

    mesh = plsc.VectorSubcoreMesh(core_axis_name="c", subcore_axis_name="s")

    @functools.partial(
        pl.kernel,
        mesh=mesh,
        out_type=jax.ShapeDtypeStruct((T, D, B), jnp.float32),
        scratch_types=[
            *[pltpu.VMEM((BB,), jnp.int32) for _ in range(NI)],
            *[pltpu.VMEM((BB, DP), jnp.float32) for _ in range(2)],
            *[pltpu.VMEM((D, BB), jnp.float32) for _ in range(2)],
            *[pltpu.SemaphoreType.DMA for _ in range(NI)],
            *[pltpu.SemaphoreType.DMA for _ in range(4)],
        ],
        compiler_params=pltpu.CompilerParams(
            use_tc_tiling_on_sc=True, needs_layout_passes=False
        ),
    )
    def gather_kernel(table_hbm, idx_hbm, out_hbm, *bufs):
        idxq = bufs[0:NI]
        rows = bufs[NI:NI + 2]
        obuf = bufs[NI + 2:NI + 4]
        isem = bufs[NI + 4:2 * NI + 4]
        gsem = bufs[2 * NI + 4:2 * NI + 6]
        osem = bufs[2 * NI + 6:2 * NI + 8]

        wid = lax.axis_index("s") * info.num_cores + lax.axis_index("c")
        blk = wid // 2
        t_base = (wid % 2) * upw
        b0 = blk * BB

        iot = lax.iota(jnp.int32, L)
        pvec = [(iot + k) & (L - 1) for k in range(L)]

        def start_idx(u, q):
            t = t_base + u
            pltpu.async_copy(
                idx_hbm.at[pl.ds(t * B + b0, BB)], idxq[q], isem[q]
            )

        def wait_idx(q):
            pltpu.make_async_copy(
                idx_hbm.at[pl.ds(0, BB)], idxq[q], isem[q]
            ).wait()

        def fire_g(q, p):
            for c in range(BB // 128):
                pltpu.async_copy(
                    table_hbm.at[idxq[q].at[pl.ds(c * 128, 128)]],
                    rows[p].at[pl.ds(c * 128, 128)],
                    gsem[p],
                )

        def drain_gather(p):
            for c in range(BB // 128):
                pltpu.make_async_copy(
                    table_hbm.at[pl.ds(0, 128)],
                    rows[p].at[pl.ds(c * 128, 128)],
                    gsem[p],
                ).wait()

        def transpose(p):
            def bbody(bb, _):
                rrow = bb * L + iot
                for c in range(D // L):
                    for k in range(L):
                        dcol = pvec[k] + c * L
                        v = plsc.load_gather(rows[p], [rrow, dcol])
                        plsc.store_scatter(obuf[p], [dcol, rrow], v)
                return _
            lax.fori_loop(0, BB // L, bbody, None)

        def start_wb(u, p):
            t = t_base + u
            pltpu.async_copy(
                obuf[p], out_hbm.at[t, :, pl.ds(b0, BB)], osem[p]
            )

        def wait_wb(p):
            pltpu.make_async_copy(
                out_hbm.at[0, :, pl.ds(0, BB)], obuf[p], osem[p]
            ).wait()

        def unit_ops(u, p, do_start_idx, do_fire, do_wait_wb, q2, q1):
            if do_start_idx:
                start_idx(u + 2, q2)
            if do_fire:
                wait_idx(q1)
                fire_g(q1, 1 - p)
            drain_gather(p)
            if do_wait_wb:
                wait_wb(p)
            transpose(p)
            start_wb(u, p)

        start_idx(0, 0)
        start_idx(1, 1)
        wait_idx(0)
        fire_g(0, 0)
        unit_ops(0, 0, True, True, False, 2, 1)
        unit_ops(1, 1, True, True, False, 3, 2)

        def body(m, _):
            u0 = 2 + 4 * m
            for s in range(4):
                unit_ops(u0 + s, s % 2, True, True, True, s, (3 + s) % 4)
            return _

        lax.fori_loop(0, (upw - 4) // 4, body, None)

        unit_ops(upw - 2, 0, False, True, True, 0, (upw - 1) % NI)
        unit_ops(upw - 1, 1, False, False, True, 0, 0)
        wait_wb(0)
        wait_wb(1)

    out = gather_kernel(table_p, idx_flat)
    return jnp.transpose(out, (2, 0, 1))


# speedup vs baseline: 2.8151x; 1.3548x over previous
import functools

import jax
import jax.numpy as jnp
from jax import lax
from jax.experimental import pallas as pl
from jax.experimental.pallas import tpu as pltpu
from jax.experimental.pallas import tpu_sc as plsc

BB = 256
L = 16
NI = 4


def kernel(lookup, table):
    B, T = lookup.shape
    V, D = table.shape
    DP = 128

    info = plsc.get_sparse_core_info()
    NW = info.num_cores * info.num_subcores
    NBLK = B // BB
    upw = (T * NBLK) // NW

    idx_flat = jnp.transpose(lookup).reshape(-1).astype(jnp.int32)
    table_p = jnp.pad(table, ((0, 0), (0, DP - D)))

    mesh = plsc.VectorSubcoreMesh(core_axis_name="c", subcore_axis_name="s")

    @functools.partial(
        pl.kernel,
        mesh=mesh,
        out_type=jax.ShapeDtypeStruct((T, D, B), jnp.float32),
        scratch_types=[
            *[pltpu.VMEM((BB,), jnp.int32) for _ in range(NI)],
            *[pltpu.VMEM((BB, DP), jnp.float32) for _ in range(2)],
            *[pltpu.VMEM((D, BB), jnp.float32) for _ in range(2)],
            *[pltpu.SemaphoreType.DMA for _ in range(NI)],
            *[pltpu.SemaphoreType.DMA for _ in range(4)],
        ],
        compiler_params=pltpu.CompilerParams(
            use_tc_tiling_on_sc=True, needs_layout_passes=False
        ),
    )
    def gather_kernel(table_hbm, idx_hbm, out_hbm, *bufs):
        idxq = bufs[0:NI]
        rows = bufs[NI:NI + 2]
        obuf = bufs[NI + 2:NI + 4]
        isem = bufs[NI + 4:2 * NI + 4]
        gsem = bufs[2 * NI + 4:2 * NI + 6]
        osem = bufs[2 * NI + 6:2 * NI + 8]

        wid = lax.axis_index("s") * info.num_cores + lax.axis_index("c")
        blk = wid // 2
        t_base = (wid % 2) * upw
        b0 = blk * BB

        iot = lax.iota(jnp.int32, L)
        pvec = [(iot + k) & (L - 1) for k in range(L)]

        def start_idx(u, q):
            t = t_base + u
            pltpu.async_copy(
                idx_hbm.at[pl.ds(t * B + b0, BB)], idxq[q], isem[q]
            )

        def wait_idx(q):
            pltpu.make_async_copy(
                idx_hbm.at[pl.ds(0, BB)], idxq[q], isem[q]
            ).wait()

        def fire_g(q, p):
            for c in range(BB // 128):
                pltpu.async_copy(
                    table_hbm.at[idxq[q].at[pl.ds(c * 128, 128)]],
                    rows[p].at[pl.ds(c * 128, 128)],
                    gsem[p],
                )

        def drain_gather(p):
            pltpu.make_async_copy(
                table_hbm.at[pl.ds(0, BB)], rows[p], gsem[p]
            ).wait()

        def transpose(p):
            def bbody(bb, _):
                for half in range(2):
                    rrow = (bb * 2 + half) * L + iot
                    for c in range(D // L):
                        for k in range(L):
                            dcol = pvec[k] + c * L
                            v = plsc.load_gather(rows[p], [rrow, dcol])
                            plsc.store_scatter(obuf[p], [dcol, rrow], v)
                return _
            lax.fori_loop(0, BB // L // 2, bbody, None)

        def start_wb(u, p):
            t = t_base + u
            pltpu.async_copy(
                obuf[p], out_hbm.at[t, :, pl.ds(b0, BB)], osem[p]
            )

        def wait_wb(p):
            pltpu.make_async_copy(
                out_hbm.at[0, :, pl.ds(0, BB)], obuf[p], osem[p]
            ).wait()

        def unit_ops(u, p, do_start_idx, do_fire, do_wait_wb, q2, q1):
            if do_start_idx:
                start_idx(u + 2, q2)
            if do_fire:
                wait_idx(q1)
                fire_g(q1, 1 - p)
            drain_gather(p)
            if do_wait_wb:
                wait_wb(p)
            transpose(p)
            start_wb(u, p)

        start_idx(0, 0)
        start_idx(1, 1)
        wait_idx(0)
        fire_g(0, 0)
        unit_ops(0, 0, True, True, False, 2, 1)
        unit_ops(1, 1, True, True, False, 3, 2)

        def body(m, _):
            u0 = 2 + 4 * m
            for s in range(4):
                unit_ops(u0 + s, s % 2, True, True, True, s, (3 + s) % 4)
            return _

        lax.fori_loop(0, (upw - 4) // 4, body, None)

        unit_ops(upw - 2, 0, False, True, True, 0, (upw - 1) % NI)
        unit_ops(upw - 1, 1, False, False, True, 0, 0)
        wait_wb(0)
        wait_wb(1)

    out = gather_kernel(table_p, idx_flat)
    return jnp.transpose(out, (2, 0, 1))
